# Initial kernel scaffold; baseline (speedup 1.0000x reference)
import jax
import jax.numpy as jnp
from jax.experimental import pallas as pl


def kernel(anchor, positive, negative):
    raise NotImplementedError("write your pallas kernel here")


# speedup vs baseline: 15.8325x; 15.8325x over previous
import functools

import jax
import jax.numpy as jnp
from jax.experimental import pallas as pl
from jax.experimental.pallas import tpu as pltpu

B = 128
D = 64
K = B // 2
MARGIN = 1.0


def _loss_kernel(a_ref, p_ref, n_ref, out_ref, vt_ref):
    a = a_ref[...]
    p = p_ref[...]
    n = n_ref[...]

    an2 = jnp.sum(a * a, axis=1, keepdims=True)
    nn2 = jnp.sum(n * n, axis=1, keepdims=True)
    dpos = a - p
    pos = jnp.sum(dpos * dpos, axis=1, keepdims=True)

    ones = jnp.ones((B, 1), dtype=jnp.float32)
    a_aug = jnp.concatenate([-2.0 * a, ones], axis=1)
    n_aug = jnp.concatenate([n, nn2], axis=1)
    nd = jax.lax.dot_general(
        a_aug, n_aug, (((1,), (1,)), ((), ())),
        preferred_element_type=jnp.float32) + an2
    nT_aug = jnp.concatenate([-2.0 * n, ones], axis=1)
    aT_aug = jnp.concatenate([a, an2], axis=1)
    ndT = jax.lax.dot_general(
        nT_aug, aT_aug, (((1,), (1,)), ((), ())),
        preferred_element_type=jnp.float32) + nn2
    vt_ref[...] = ndT

    term1 = jnp.sum(jnp.maximum(MARGIN + pos - ndT, 0.0),
                    axis=(0, 1), keepdims=True)

    sub = jax.lax.broadcasted_iota(jnp.int32, (B, B), 0)

    def body(jp, rank):
        pivot = vt_ref[pl.ds(jp, 1), :]
        lt = (pivot < ndT).astype(jnp.float32)
        eq = jnp.logical_and(pivot == ndT, sub > jp).astype(jnp.float32)
        return rank + lt + eq

    rank = jax.lax.fori_loop(0, B, body, jnp.zeros((B, B), jnp.float32))
    maskT = (rank < float(K)).astype(jnp.float32)
    count = jnp.sum(maskT, axis=1, keepdims=True)

    clip2 = jnp.maximum(MARGIN + pos - nd, 0.0)
    colsum2 = jnp.sum(clip2, axis=0, keepdims=True)
    term2 = jax.lax.dot_general(
        colsum2, count, (((1,), (0,)), ((), ())),
        preferred_element_type=jnp.float32)

    out_ref[...] = term1 / (B * B) + term2 / (B * B * K)


@functools.partial(jax.jit)
def kernel(anchor, positive, negative):
    out = pl.pallas_call(
        _loss_kernel,
        out_shape=jax.ShapeDtypeStruct((1, 1), jnp.float32),
        scratch_shapes=[pltpu.VMEM((B, B), jnp.float32)],
    )(anchor, positive, negative)
    return out[0, 0]
